# baseline (speedup 1.0000x reference)
import functools
import jax
import jax.numpy as jnp
from jax import lax
from jax.experimental import pallas as pl
from jax.experimental.pallas import tpu as pltpu
from jax.experimental.pallas import tpu_sc as plsc

B = 16384
ENT = 64
NNODES = 1000000
NREL = 1000
NREL_PAD = 1024
NC = 2
NS = 16
NW = NC * NS
PER_W = B // NW
CHUNK = 128
NCHUNK = PER_W // CHUNK

RW = 8192
TPW = 52
CSPLIT = NW * TPW * 128
TC_OFF = CSPLIT // RW
RSTEPS = (NNODES + RW - 1) // RW - TC_OFF


def _reduce_body(hw, hpw, tw, tpw, dh, sh, dt, st):
    h = hw[...]
    t = tw[...]
    dh[...] = jnp.sum(hpw[...] * h, axis=0)
    sh[...] = jnp.sum(h, axis=0)
    dt[...] = jnp.sum(tpw[...] * t, axis=0)
    st[...] = jnp.sum(t, axis=0)


def _stats_body(rw, rpw, P, Q, S, R2, Sr):
    col = jax.lax.broadcasted_iota(jnp.int32, (1, NREL_PAD), 1)
    m = (col < NREL).astype(jnp.float32)
    r = rw[...] * m
    rp = rpw[...] * m
    P[...] = jnp.sum(rp * rp, axis=0)
    Q[...] = jnp.sum(rp * r, axis=0)
    S[...] = jnp.sum(rp, axis=0)
    R2[...] = jnp.sum(r * r, axis=0)
    Sr[...] = jnp.sum(r, axis=0)


def _stream_body(hw_hbm, hpw_hbm, tw_hbm, tpw_hbm,
                 dh_hbm, sh_hbm, dt_hbm, st_hbm,
                 bhw, bhpw, btw, btpw, dh_v, sh_v, dt_v, st_v, sem):
    cid = lax.axis_index("c")
    sid = lax.axis_index("s")
    wid = sid * NC + cid
    col0 = wid * (TPW * 128)

    def tile_step(tc, _):
        col = col0 + tc * 128
        cp1 = pltpu.async_copy(hw_hbm.at[:, pl.ds(col, 128)], bhw, sem)
        cp2 = pltpu.async_copy(hpw_hbm.at[:, pl.ds(col, 128)], bhpw, sem)
        cp3 = pltpu.async_copy(tw_hbm.at[:, pl.ds(col, 128)], btw, sem)
        cp4 = pltpu.async_copy(tpw_hbm.at[:, pl.ds(col, 128)], btpw, sem)
        cp1.wait()
        cp2.wait()
        cp3.wait()
        cp4.wait()
        for g in range(8):
            sl = pl.ds(g * 16, 16)

            def jstep(j, carry):
                ad, as_, bd, bs = carry
                vh = bhw[j, sl]
                vhp = bhpw[j, sl]
                vt = btw[j, sl]
                vtp = btpw[j, sl]
                return (ad + vhp * vh, as_ + vh, bd + vtp * vt, bs + vt)

            z = jnp.zeros((16,), jnp.float32)
            ad, as_, bd, bs = lax.fori_loop(0, ENT, jstep, (z, z, z, z))
            out_sl = pl.ds(tc * 128 + g * 16, 16)
            dh_v[out_sl] = ad
            sh_v[out_sl] = as_
            dt_v[out_sl] = bd
            st_v[out_sl] = bs
        return 0

    lax.fori_loop(0, TPW, tile_step, 0)
    dst = pl.ds(col0, TPW * 128)
    pltpu.sync_copy(dh_v, dh_hbm.at[dst])
    pltpu.sync_copy(sh_v, sh_hbm.at[dst])
    pltpu.sync_copy(dt_v, dt_hbm.at[dst])
    pltpu.sync_copy(st_v, st_hbm.at[dst])


def _score_body(hidx_hbm, tidx_hbm, ridx_hbm,
                dhl_hbm, shl_hbm, dtl_hbm, stl_hbm,
                dh_hbm, sh_hbm, dt_hbm, st_hbm,
                P_hbm, Q_hbm, S_hbm, R2_hbm, Sr_hbm, out_hbm,
                P_v, Q_v, S_v, R2_v, Sr_v, hidx_v, tidx_v, ridx_v,
                dhl_v, shl_v, dtl_v, stl_v,
                dh_v, sh_v, dt_v, st_v, hlo_v, tlo_v, hhi_v, thi_v,
                out_buf, sem):
    cid = lax.axis_index("c")
    sid = lax.axis_index("s")
    wid = sid * NC + cid
    pltpu.sync_copy(P_hbm, P_v)
    pltpu.sync_copy(Q_hbm, Q_v)
    pltpu.sync_copy(S_hbm, S_v)
    pltpu.sync_copy(R2_hbm, R2_v)
    pltpu.sync_copy(Sr_hbm, Sr_v)
    for chunk in range(NCHUNK):
        base = wid * PER_W + chunk * CHUNK
        pltpu.sync_copy(hidx_hbm.at[pl.ds(base, CHUNK)], hidx_v)
        pltpu.sync_copy(tidx_hbm.at[pl.ds(base, CHUNK)], tidx_v)
        pltpu.sync_copy(ridx_hbm.at[pl.ds(base, CHUNK)], ridx_v)
        for g in range(CHUNK // 16):
            sl = pl.ds(g * 16, 16)
            hi = hidx_v[sl]
            ti = tidx_v[sl]
            hlo_v[sl] = jnp.minimum(hi, CSPLIT - 1)
            tlo_v[sl] = jnp.minimum(ti, CSPLIT - 1)
            hhi_v[sl] = jnp.maximum(hi - CSPLIT, 0)
            thi_v[sl] = jnp.maximum(ti - CSPLIT, 0)
        cps = [
            pltpu.async_copy(dhl_hbm.at[hlo_v], dhl_v, sem),
            pltpu.async_copy(shl_hbm.at[hlo_v], shl_v, sem),
            pltpu.async_copy(dtl_hbm.at[tlo_v], dtl_v, sem),
            pltpu.async_copy(stl_hbm.at[tlo_v], stl_v, sem),
            pltpu.async_copy(dh_hbm.at[hhi_v], dh_v, sem),
            pltpu.async_copy(sh_hbm.at[hhi_v], sh_v, sem),
            pltpu.async_copy(dt_hbm.at[thi_v], dt_v, sem),
            pltpu.async_copy(st_hbm.at[thi_v], st_v, sem),
        ]
        for cp in cps:
            cp.wait()
        for g in range(CHUNK // 16):
            sl = pl.ds(g * 16, 16)
            in_hi_h = hidx_v[sl] >= CSPLIT
            in_hi_t = tidx_v[sl] >= CSPLIT
            aa_h = jnp.where(in_hi_h, dh_v[sl], dhl_v[sl])
            cc_h = jnp.where(in_hi_h, sh_v[sl], shl_v[sl])
            aa_t = jnp.where(in_hi_t, dt_v[sl], dtl_v[sl])
            cc_t = jnp.where(in_hi_t, st_v[sl], stl_v[sl])
            aa = aa_h - aa_t
            cc = cc_h - cc_t
            rvec = ridx_v[sl]
            Pv = plsc.load_gather(P_v, [rvec])
            Qv = plsc.load_gather(Q_v, [rvec])
            Sv = plsc.load_gather(S_v, [rvec])
            R2v = plsc.load_gather(R2_v, [rvec])
            Srv = plsc.load_gather(Sr_v, [rvec])
            score = (aa * aa * Pv + 2.0 * aa * Qv + 2.0 * aa * cc * Sv
                     + R2v + 2.0 * cc * Srv + 64.0 * cc * cc)
            out_buf[sl] = score
        pltpu.sync_copy(out_buf, out_hbm.at[pl.ds(base, CHUNK)])


@jax.jit
def _transd(head_indices, tail_indices, relation_indices,
            head_w, head_p_w, tail_w, tail_p_w, rel_w, rel_p_w):
    hw_t = head_w.T
    hpw_t = head_p_w.T
    tw_t = tail_w.T
    tpw_t = tail_p_w.T

    mesh = plsc.VectorSubcoreMesh(core_axis_name="c", subcore_axis_name="s")
    lovec = jax.ShapeDtypeStruct((CSPLIT,), jnp.float32)
    stream = pl.kernel(
        _stream_body,
        out_type=[lovec] * 4,
        mesh=mesh,
        compiler_params=pltpu.CompilerParams(use_tc_tiling_on_sc=True),
        scratch_types=[
            pltpu.VMEM((ENT, 128), jnp.float32),
            pltpu.VMEM((ENT, 128), jnp.float32),
            pltpu.VMEM((ENT, 128), jnp.float32),
            pltpu.VMEM((ENT, 128), jnp.float32),
            pltpu.VMEM((TPW * 128,), jnp.float32),
            pltpu.VMEM((TPW * 128,), jnp.float32),
            pltpu.VMEM((TPW * 128,), jnp.float32),
            pltpu.VMEM((TPW * 128,), jnp.float32),
            pltpu.SemaphoreType.DMA,
        ],
    )
    dhl, shl, dtl, stl = stream(hw_t, hpw_t, tw_t, tpw_t)

    vec = jax.ShapeDtypeStruct((NNODES - CSPLIT,), jnp.float32)
    dh, sh, dt, st = pl.pallas_call(
        _reduce_body,
        grid=(RSTEPS,),
        in_specs=[pl.BlockSpec((ENT, RW), lambda c: (0, c + TC_OFF))] * 4,
        out_specs=[pl.BlockSpec((RW,), lambda c: (c,))] * 4,
        out_shape=[vec] * 4,
    )(hw_t, hpw_t, tw_t, tpw_t)

    rvec = jax.ShapeDtypeStruct((NREL_PAD,), jnp.float32)
    P, Q, S, R2, Sr = pl.pallas_call(
        _stats_body,
        grid=(1,),
        in_specs=[pl.BlockSpec((ENT, NREL_PAD), lambda c: (0, 0))] * 2,
        out_specs=[pl.BlockSpec((NREL_PAD,), lambda c: (0,))] * 5,
        out_shape=[rvec] * 5,
    )(rel_w.T, rel_p_w.T)

    run = pl.kernel(
        _score_body,
        out_type=jax.ShapeDtypeStruct((B,), jnp.float32),
        mesh=mesh,
        compiler_params=pltpu.CompilerParams(needs_layout_passes=False),
        scratch_types=[
            pltpu.VMEM((NREL_PAD,), jnp.float32),
            pltpu.VMEM((NREL_PAD,), jnp.float32),
            pltpu.VMEM((NREL_PAD,), jnp.float32),
            pltpu.VMEM((NREL_PAD,), jnp.float32),
            pltpu.VMEM((NREL_PAD,), jnp.float32),
            pltpu.VMEM((CHUNK,), jnp.int32),
            pltpu.VMEM((CHUNK,), jnp.int32),
            pltpu.VMEM((CHUNK,), jnp.int32),
            pltpu.VMEM((CHUNK,), jnp.float32),
            pltpu.VMEM((CHUNK,), jnp.float32),
            pltpu.VMEM((CHUNK,), jnp.float32),
            pltpu.VMEM((CHUNK,), jnp.float32),
            pltpu.VMEM((CHUNK,), jnp.float32),
            pltpu.VMEM((CHUNK,), jnp.float32),
            pltpu.VMEM((CHUNK,), jnp.float32),
            pltpu.VMEM((CHUNK,), jnp.float32),
            pltpu.VMEM((CHUNK,), jnp.int32),
            pltpu.VMEM((CHUNK,), jnp.int32),
            pltpu.VMEM((CHUNK,), jnp.int32),
            pltpu.VMEM((CHUNK,), jnp.int32),
            pltpu.VMEM((CHUNK,), jnp.float32),
            pltpu.SemaphoreType.DMA,
        ],
    )
    return run(head_indices, tail_indices, relation_indices,
               dhl, shl, dtl, stl, dh, sh, dt, st, P, Q, S, R2, Sr)


def kernel(head_indices, tail_indices, relation_indices,
           head_w, head_p_w, tail_w, tail_p_w, rel_w, rel_p_w):
    return _transd(head_indices.astype(jnp.int32),
                   tail_indices.astype(jnp.int32),
                   relation_indices.astype(jnp.int32),
                   head_w, head_p_w, tail_w, tail_p_w, rel_w, rel_p_w)


# speedup vs baseline: 1.1691x; 1.1691x over previous
import functools
import jax
import jax.numpy as jnp
from jax import lax
from jax.experimental import pallas as pl
from jax.experimental.pallas import tpu as pltpu
from jax.experimental.pallas import tpu_sc as plsc

B = 16384
ENT = 64
NNODES = 1000000
NREL = 1000
NREL_PAD = 1024
NC = 2
NS = 16
NW = NC * NS
PER_W = B // NW
CHUNK = 128
NCHUNK = PER_W // CHUNK
RW = 16384
RSTEPS = (NNODES + RW - 1) // RW


def _reduce_body(hw, hpw, tw, tpw, dh, sh, dt, st):
    h = hw[...]
    t = tw[...]
    dh[...] = jnp.sum(hpw[...] * h, axis=0)
    sh[...] = jnp.sum(h, axis=0)
    dt[...] = jnp.sum(tpw[...] * t, axis=0)
    st[...] = jnp.sum(t, axis=0)


def _stats_body(rw, rpw, P, Q, S, R2, Sr):
    col = jax.lax.broadcasted_iota(jnp.int32, (1, NREL_PAD), 1)
    m = (col < NREL).astype(jnp.float32)
    r = rw[...] * m
    rp = rpw[...] * m
    P[...] = jnp.sum(rp * rp, axis=0)
    Q[...] = jnp.sum(rp * r, axis=0)
    S[...] = jnp.sum(rp, axis=0)
    R2[...] = jnp.sum(r * r, axis=0)
    Sr[...] = jnp.sum(r, axis=0)


def _score_body(hidx_hbm, tidx_hbm, ridx_hbm, dh_hbm, sh_hbm, dt_hbm, st_hbm,
                P_hbm, Q_hbm, S_hbm, R2_hbm, Sr_hbm, out_hbm,
                P_v, Q_v, S_v, R2_v, Sr_v, hidx_v, tidx_v, ridx_v,
                dh_v, sh_v, dt_v, st_v, out_buf, sem):
    cid = lax.axis_index("c")
    sid = lax.axis_index("s")
    wid = sid * NC + cid
    pltpu.sync_copy(P_hbm, P_v)
    pltpu.sync_copy(Q_hbm, Q_v)
    pltpu.sync_copy(S_hbm, S_v)
    pltpu.sync_copy(R2_hbm, R2_v)
    pltpu.sync_copy(Sr_hbm, Sr_v)
    for chunk in range(NCHUNK):
        base = wid * PER_W + chunk * CHUNK
        pltpu.sync_copy(hidx_hbm.at[pl.ds(base, CHUNK)], hidx_v)
        pltpu.sync_copy(tidx_hbm.at[pl.ds(base, CHUNK)], tidx_v)
        pltpu.sync_copy(ridx_hbm.at[pl.ds(base, CHUNK)], ridx_v)
        cp1 = pltpu.async_copy(dh_hbm.at[hidx_v], dh_v, sem)
        cp2 = pltpu.async_copy(sh_hbm.at[hidx_v], sh_v, sem)
        cp3 = pltpu.async_copy(dt_hbm.at[tidx_v], dt_v, sem)
        cp4 = pltpu.async_copy(st_hbm.at[tidx_v], st_v, sem)
        cp1.wait()
        cp2.wait()
        cp3.wait()
        cp4.wait()
        for g in range(CHUNK // 16):
            sl = pl.ds(g * 16, 16)
            aa = dh_v[sl] - dt_v[sl]
            cc = sh_v[sl] - st_v[sl]
            rvec = ridx_v[sl]
            Pv = plsc.load_gather(P_v, [rvec])
            Qv = plsc.load_gather(Q_v, [rvec])
            Sv = plsc.load_gather(S_v, [rvec])
            R2v = plsc.load_gather(R2_v, [rvec])
            Srv = plsc.load_gather(Sr_v, [rvec])
            score = (aa * aa * Pv + 2.0 * aa * Qv + 2.0 * aa * cc * Sv
                     + R2v + 2.0 * cc * Srv + 64.0 * cc * cc)
            out_buf[sl] = score
        pltpu.sync_copy(out_buf, out_hbm.at[pl.ds(base, CHUNK)])


@jax.jit
def _transd(head_indices, tail_indices, relation_indices,
            head_w, head_p_w, tail_w, tail_p_w, rel_w, rel_p_w):
    hw_t = head_w.T
    hpw_t = head_p_w.T
    tw_t = tail_w.T
    tpw_t = tail_p_w.T

    vec = jax.ShapeDtypeStruct((NNODES,), jnp.float32)
    dh, sh, dt, st = pl.pallas_call(
        _reduce_body,
        grid=(RSTEPS,),
        in_specs=[pl.BlockSpec((ENT, RW), lambda c: (0, c))] * 4,
        out_specs=[pl.BlockSpec((RW,), lambda c: (c,))] * 4,
        out_shape=[vec] * 4,
    )(hw_t, hpw_t, tw_t, tpw_t)

    rvec = jax.ShapeDtypeStruct((NREL_PAD,), jnp.float32)
    P, Q, S, R2, Sr = pl.pallas_call(
        _stats_body,
        grid=(1,),
        in_specs=[pl.BlockSpec((ENT, NREL_PAD), lambda c: (0, 0))] * 2,
        out_specs=[pl.BlockSpec((NREL_PAD,), lambda c: (0,))] * 5,
        out_shape=[rvec] * 5,
    )(rel_w.T, rel_p_w.T)

    mesh = plsc.VectorSubcoreMesh(core_axis_name="c", subcore_axis_name="s")
    run = pl.kernel(
        _score_body,
        out_type=jax.ShapeDtypeStruct((B,), jnp.float32),
        mesh=mesh,
        compiler_params=pltpu.CompilerParams(needs_layout_passes=False),
        scratch_types=[
            pltpu.VMEM((NREL_PAD,), jnp.float32),
            pltpu.VMEM((NREL_PAD,), jnp.float32),
            pltpu.VMEM((NREL_PAD,), jnp.float32),
            pltpu.VMEM((NREL_PAD,), jnp.float32),
            pltpu.VMEM((NREL_PAD,), jnp.float32),
            pltpu.VMEM((CHUNK,), jnp.int32),
            pltpu.VMEM((CHUNK,), jnp.int32),
            pltpu.VMEM((CHUNK,), jnp.int32),
            pltpu.VMEM((CHUNK,), jnp.float32),
            pltpu.VMEM((CHUNK,), jnp.float32),
            pltpu.VMEM((CHUNK,), jnp.float32),
            pltpu.VMEM((CHUNK,), jnp.float32),
            pltpu.VMEM((CHUNK,), jnp.float32),
            pltpu.SemaphoreType.DMA,
        ],
    )
    return run(head_indices, tail_indices, relation_indices,
               dh, sh, dt, st, P, Q, S, R2, Sr)


def kernel(head_indices, tail_indices, relation_indices,
           head_w, head_p_w, tail_w, tail_p_w, rel_w, rel_p_w):
    return _transd(head_indices.astype(jnp.int32),
                   tail_indices.astype(jnp.int32),
                   relation_indices.astype(jnp.int32),
                   head_w, head_p_w, tail_w, tail_p_w, rel_w, rel_p_w)
